# baseline (speedup 1.0000x reference)
import functools

import jax
import jax.numpy as jnp
from jax import lax
from jax.experimental import pallas as pl
from jax.experimental.pallas import tpu as pltpu
from jax.experimental.pallas import tpu_sc as plsc

N = 10000
D = 128
E = 320000
L = 2
LEAKY = 0.2

NC = 2
NS = 16
LANE = 16
DH = D // NC
CB = 128
SG = 8
NSG = 20
CH = SG * NSG
E_PAD = NS * CH * CB
RPT = N // NS
_STAGE = [(0, 128), (128, 128), (256, 128), (384, 128), (512, 113)]


def _conv_body(x_hbm, meta_hbm, y_hbm, x_sh, t_sh, rb, mv2, gsem, ssem, msem):
    c = lax.axis_index("c")
    s = lax.axis_index("s")
    r0 = s * RPT
    cols_sl = pl.ds(c * DH, DH)

    def _zero_rb0():
        zv = jnp.zeros((LANE,), jnp.float32)

        def zr(i, carry):
            for kk in range(DH // LANE):
                rb[0, i, pl.ds(kk * LANE, LANE)] = zv
            return carry
        lax.fori_loop(0, CB, zr, jnp.int32(0))

    def _fill_zero(dst_sh):
        for st, sz in _STAGE:
            pltpu.sync_copy(rb.at[0, pl.ds(0, sz)],
                            dst_sh.at[pl.ds(r0 + st, sz)])

    def _scale(p, k, b):
        pv = jnp.full((LANE,), p, jnp.int32)
        twov = jnp.full((LANE,), 2, jnp.int32)
        kv = jnp.full((LANE,), k, jnp.int32)

        def edge(e, carry):
            wv = plsc.bitcast(
                plsc.load_gather(
                    mv2, [pv, twov, kv, jnp.full((LANE,), e, jnp.int32)]),
                jnp.float32)
            for kk in range(DH // LANE):
                sl = pl.ds(kk * LANE, LANE)
                rb[b, e, sl] = rb[b, e, sl] * wv
            return carry
        lax.fori_loop(0, CB, edge, jnp.int32(0), unroll=4)

    def _pass(src_sh, dst_sh, gi, si):
        pltpu.sync_copy(meta_hbm.at[s, 0], mv2.at[0])
        pltpu.async_copy(src_sh.at[mv2.at[0, gi, 0]], rb.at[0], gsem.at[0])

        def outer(i, carry):
            for p in range(2):
                sg = i * 2 + p
                for k in range(SG):
                    b = k % 4
                    g = sg * SG + k
                    slot = (k + 2) % 4
                    if k < SG - 1:
                        pltpu.async_copy(src_sh.at[mv2.at[p, gi, k + 1]],
                                         rb.at[(k + 1) % 4],
                                         gsem.at[(k + 1) % 4])
                    else:
                        @pl.when(sg + 1 < NSG)
                        def _():
                            pltpu.make_async_copy(meta_hbm.at[s, 0],
                                                  mv2.at[1 - p],
                                                  msem.at[1 - p]).wait()
                            pltpu.async_copy(src_sh.at[mv2.at[1 - p, gi, 0]],
                                             rb.at[0], gsem.at[0])
                    pltpu.make_async_copy(src_sh.at[mv2.at[p, gi, k]],
                                          rb.at[b], gsem.at[b]).wait()
                    _scale(p, k, b)
                    pltpu.async_copy(rb.at[b], dst_sh.at[mv2.at[p, si, k]],
                                     ssem.at[b], add=True)

                    @pl.when(g >= 2)
                    def _():
                        pltpu.make_async_copy(
                            rb.at[slot], dst_sh.at[mv2.at[p, si, k]],
                            ssem.at[slot]).wait()
                    if k == 2:
                        @pl.when(sg + 1 < NSG)
                        def _():
                            pltpu.async_copy(meta_hbm.at[s, sg + 1],
                                             mv2.at[1 - p], msem.at[1 - p])
            return carry
        lax.fori_loop(0, NSG // 2, outer, jnp.int32(0))
        pltpu.make_async_copy(rb.at[2], dst_sh.at[mv2.at[1, si, SG - 2]],
                              ssem.at[2]).wait()
        pltpu.make_async_copy(rb.at[3], dst_sh.at[mv2.at[1, si, SG - 1]],
                              ssem.at[3]).wait()

    pltpu.sync_copy(x_hbm.at[pl.ds(r0, RPT), cols_sl],
                    x_sh.at[pl.ds(r0, RPT)])
    _zero_rb0()
    _fill_zero(t_sh)
    plsc.subcore_barrier()

    _pass(x_sh, t_sh, 0, 1)
    plsc.subcore_barrier()

    _zero_rb0()
    _fill_zero(x_sh)
    plsc.subcore_barrier()

    _pass(t_sh, x_sh, 1, 0)
    plsc.subcore_barrier()

    for st, sz in _STAGE:
        pltpu.sync_copy(x_sh.at[pl.ds(r0 + st, sz)], rb.at[0, pl.ds(0, sz)])
        pltpu.sync_copy(rb.at[0, pl.ds(0, sz)],
                        y_hbm.at[pl.ds(r0 + st, sz), cols_sl])


_conv = pl.kernel(
    _conv_body,
    out_type=jax.ShapeDtypeStruct((N, D), jnp.float32),
    mesh=plsc.VectorSubcoreMesh(core_axis_name="c", subcore_axis_name="s",
                                num_cores=NC, num_subcores=NS),
    scratch_types=[
        pltpu.VMEM_SHARED((N, DH), jnp.float32),
        pltpu.VMEM_SHARED((N, DH), jnp.float32),
        pltpu.VMEM((4, CB, DH), jnp.float32),
        pltpu.VMEM((2, 3, SG, CB), jnp.int32),
        pltpu.SemaphoreType.DMA((4,)),
        pltpu.SemaphoreType.DMA((4,)),
        pltpu.SemaphoreType.DMA((2,)),
    ],
    compiler_params=pltpu.CompilerParams(use_tc_tiling_on_sc=False,
                                         needs_layout_passes=False),
)


def _ln_res_kernel(y_ref, res_ref, g_ref, b_ref, o_ref, *, leaky):
    y = y_ref[...]
    if leaky:
        y = jnp.where(y >= 0, y, LEAKY * y)
    mu = jnp.mean(y, axis=-1, keepdims=True)
    var = jnp.mean((y - mu) ** 2, axis=-1, keepdims=True)
    o_ref[...] = (y - mu) * lax.rsqrt(var + 1e-5) * g_ref[...] + b_ref[...] + res_ref[...]


def _ln_res(y, res, g, b, leaky):
    return pl.pallas_call(
        functools.partial(_ln_res_kernel, leaky=leaky),
        out_shape=jax.ShapeDtypeStruct((N, D), jnp.float32),
        grid=(10,),
        in_specs=[
            pl.BlockSpec((N // 10, D), lambda i: (i, 0)),
            pl.BlockSpec((N // 10, D), lambda i: (i, 0)),
            pl.BlockSpec((1, D), lambda i: (0, 0)),
            pl.BlockSpec((1, D), lambda i: (0, 0)),
        ],
        out_specs=pl.BlockSpec((N // 10, D), lambda i: (i, 0)),
    )(y, res, g[None], b[None])


def kernel(embs, edge_index, edge_weight, ln_gamma, ln_beta):
    pad = E_PAD - E
    row4 = jnp.pad(edge_index[0], (0, pad)).reshape(NS, NSG, SG, CB)
    col4 = jnp.pad(edge_index[1], (0, pad)).reshape(NS, NSG, SG, CB)
    w4 = lax.bitcast_convert_type(
        jnp.pad(edge_weight, (0, pad)), jnp.int32).reshape(NS, NSG, SG, CB)
    meta = jnp.stack([row4, col4, w4], axis=2)
    h = embs
    for k in range(L):
        y = _conv(h, meta)
        h = _ln_res(y, embs, ln_gamma[k], ln_beta[k], leaky=(k != L - 1))
    return h


# speedup vs baseline: 1.0225x; 1.0225x over previous
import functools

import jax
import jax.numpy as jnp
from jax import lax
from jax.experimental import pallas as pl
from jax.experimental.pallas import tpu as pltpu
from jax.experimental.pallas import tpu_sc as plsc

N = 10000
D = 128
E = 320000
L = 2
LEAKY = 0.2

NC = 2
NS = 16
LANE = 16
DH = D // NC
CB = 128
SG = 8
NSG = 20
CH = SG * NSG
E_PAD = NS * CH * CB
RPT = N // NS
_STAGE = [(0, 128), (128, 128), (256, 128), (384, 128), (512, 113)]


def _conv_body(x_hbm, meta_hbm, y_hbm, x_sh, t_sh, rb, mv2, gsem, ssem, msem):
    c = lax.axis_index("c")
    s = lax.axis_index("s")
    r0 = s * RPT
    cols_sl = pl.ds(c * DH, DH)

    def _zero_rb0():
        zv = jnp.zeros((LANE,), jnp.float32)

        def zr(i, carry):
            for kk in range(DH // LANE):
                rb[0, i, pl.ds(kk * LANE, LANE)] = zv
            return carry
        lax.fori_loop(0, CB, zr, jnp.int32(0))

    def _fill_zero(dst_sh):
        for st, sz in _STAGE:
            pltpu.sync_copy(rb.at[0, pl.ds(0, sz)],
                            dst_sh.at[pl.ds(r0 + st, sz)])

    def _scale(p, k, b):
        pv = jnp.full((LANE,), p, jnp.int32)
        twov = jnp.full((LANE,), 2, jnp.int32)
        kv = jnp.full((LANE,), k, jnp.int32)

        def edge(e, carry):
            wv = plsc.bitcast(
                plsc.load_gather(
                    mv2, [pv, twov, kv, jnp.full((LANE,), e, jnp.int32)]),
                jnp.float32)
            for kk in range(DH // LANE):
                sl = pl.ds(kk * LANE, LANE)
                rb[b, e, sl] = rb[b, e, sl] * wv
            return carry
        lax.fori_loop(0, CB, edge, jnp.int32(0), unroll=2)

    def _pass(src_sh, dst_sh, gi, si):
        pltpu.sync_copy(meta_hbm.at[s, 0], mv2.at[0])
        pltpu.async_copy(src_sh.at[mv2.at[0, gi, 0]], rb.at[0], gsem.at[0])

        def outer(i, carry):
            for p in range(2):
                sg = i * 2 + p
                for k in range(SG):
                    b = k % 4
                    g = sg * SG + k
                    slot = (k + 2) % 4
                    if k < SG - 1:
                        pltpu.async_copy(src_sh.at[mv2.at[p, gi, k + 1]],
                                         rb.at[(k + 1) % 4],
                                         gsem.at[(k + 1) % 4])
                    else:
                        @pl.when(sg + 1 < NSG)
                        def _():
                            pltpu.make_async_copy(meta_hbm.at[s, 0],
                                                  mv2.at[1 - p],
                                                  msem.at[1 - p]).wait()
                            pltpu.async_copy(src_sh.at[mv2.at[1 - p, gi, 0]],
                                             rb.at[0], gsem.at[0])
                    pltpu.make_async_copy(src_sh.at[mv2.at[p, gi, k]],
                                          rb.at[b], gsem.at[b]).wait()
                    _scale(p, k, b)
                    pltpu.async_copy(rb.at[b], dst_sh.at[mv2.at[p, si, k]],
                                     ssem.at[b], add=True)

                    @pl.when(g >= 2)
                    def _():
                        pltpu.make_async_copy(
                            rb.at[slot], dst_sh.at[mv2.at[p, si, k]],
                            ssem.at[slot]).wait()
                    if k == 2:
                        @pl.when(sg + 1 < NSG)
                        def _():
                            pltpu.async_copy(meta_hbm.at[s, sg + 1],
                                             mv2.at[1 - p], msem.at[1 - p])
            return carry
        lax.fori_loop(0, NSG // 2, outer, jnp.int32(0))
        pltpu.make_async_copy(rb.at[2], dst_sh.at[mv2.at[1, si, SG - 2]],
                              ssem.at[2]).wait()
        pltpu.make_async_copy(rb.at[3], dst_sh.at[mv2.at[1, si, SG - 1]],
                              ssem.at[3]).wait()

    pltpu.sync_copy(x_hbm.at[pl.ds(r0, RPT), cols_sl],
                    x_sh.at[pl.ds(r0, RPT)])
    _zero_rb0()
    _fill_zero(t_sh)
    plsc.subcore_barrier()

    _pass(x_sh, t_sh, 0, 1)
    plsc.subcore_barrier()

    _zero_rb0()
    _fill_zero(x_sh)
    plsc.subcore_barrier()

    _pass(t_sh, x_sh, 1, 0)
    plsc.subcore_barrier()

    pltpu.sync_copy(x_sh.at[pl.ds(r0, RPT)],
                    y_hbm.at[pl.ds(r0, RPT), cols_sl])


_conv = pl.kernel(
    _conv_body,
    out_type=jax.ShapeDtypeStruct((N, D), jnp.float32),
    mesh=plsc.VectorSubcoreMesh(core_axis_name="c", subcore_axis_name="s",
                                num_cores=NC, num_subcores=NS),
    scratch_types=[
        pltpu.VMEM_SHARED((N, DH), jnp.float32),
        pltpu.VMEM_SHARED((N, DH), jnp.float32),
        pltpu.VMEM((4, CB, DH), jnp.float32),
        pltpu.VMEM((2, 3, SG, CB), jnp.int32),
        pltpu.SemaphoreType.DMA((4,)),
        pltpu.SemaphoreType.DMA((4,)),
        pltpu.SemaphoreType.DMA((2,)),
    ],
    compiler_params=pltpu.CompilerParams(use_tc_tiling_on_sc=False,
                                         needs_layout_passes=False),
)


def _ln_res_kernel(y_ref, res_ref, g_ref, b_ref, o_ref, *, leaky):
    y = y_ref[...]
    if leaky:
        y = jnp.where(y >= 0, y, LEAKY * y)
    mu = jnp.mean(y, axis=-1, keepdims=True)
    var = jnp.mean((y - mu) ** 2, axis=-1, keepdims=True)
    o_ref[...] = (y - mu) * lax.rsqrt(var + 1e-5) * g_ref[...] + b_ref[...] + res_ref[...]


def _ln_res(y, res, g, b, leaky):
    return pl.pallas_call(
        functools.partial(_ln_res_kernel, leaky=leaky),
        out_shape=jax.ShapeDtypeStruct((N, D), jnp.float32),
        grid=(10,),
        in_specs=[
            pl.BlockSpec((N // 10, D), lambda i: (i, 0)),
            pl.BlockSpec((N // 10, D), lambda i: (i, 0)),
            pl.BlockSpec((1, D), lambda i: (0, 0)),
            pl.BlockSpec((1, D), lambda i: (0, 0)),
        ],
        out_specs=pl.BlockSpec((N // 10, D), lambda i: (i, 0)),
    )(y, res, g[None], b[None])


def kernel(embs, edge_index, edge_weight, ln_gamma, ln_beta):
    pad = E_PAD - E
    row4 = jnp.pad(edge_index[0], (0, pad)).reshape(NS, NSG, SG, CB)
    col4 = jnp.pad(edge_index[1], (0, pad)).reshape(NS, NSG, SG, CB)
    w4 = lax.bitcast_convert_type(
        jnp.pad(edge_weight, (0, pad)), jnp.int32).reshape(NS, NSG, SG, CB)
    meta = jnp.stack([row4, col4, w4], axis=2)
    h = embs
    for k in range(L):
        y = _conv(h, meta)
        h = _ln_res(y, embs, ln_gamma[k], ln_beta[k], leaky=(k != L - 1))
    return h
